# Initial kernel scaffold; baseline (speedup 1.0000x reference)
import jax
import jax.numpy as jnp
from jax.experimental import pallas as pl


def kernel(waypointslocal, boundarypoints, boundarynormals):
    raise NotImplementedError("write your pallas kernel here")


# speedup vs baseline: 1.5292x; 1.5292x over previous
import functools

import jax
import jax.numpy as jnp
from jax import lax
from jax.experimental import pallas as pl
from jax.experimental.pallas import tpu as pltpu
from jax.experimental.pallas import tpu_sc as plsc


def _make_argmin(B, W, N, D, BN):
    NB = N // BN

    def body(wp_ref, bp_ref, idx_ref, idxg_ref, minval, minidx):
        b = pl.program_id(0)
        nb = pl.program_id(1)

        @pl.when(nb == 0)
        def _init():
            minval[...] = jnp.full((W,), jnp.inf, jnp.float32)
            minidx[...] = jnp.zeros((W,), jnp.int32)

        wp = wp_ref[0]
        bp = bp_ref[0]
        w2 = jnp.sum(wp * wp, axis=1)
        b2 = jnp.sum(bp * bp, axis=1)
        cross = lax.dot_general(
            wp, bp, (((1,), (1,)), ((), ())),
            preferred_element_type=jnp.float32)
        s = (w2[:, None] + b2[None, :]) - 2.0 * cross
        local_min = jnp.min(s, axis=1)
        ids = lax.broadcasted_iota(jnp.int32, (W, BN), 1)
        masked = jnp.where(s == local_min[:, None], ids, N)
        local_idx = jnp.min(masked, axis=1) + nb * BN
        better = local_min < minval[...]
        minidx[...] = jnp.where(better, local_idx, minidx[...])
        minval[...] = jnp.where(better, local_min, minval[...])

        @pl.when(nb == NB - 1)
        def _fin():
            idx_ref[0, 0] = minidx[...]
            idxg_ref[0, 0] = minidx[...] + b * N

    return pl.pallas_call(
        body,
        grid=(B, NB),
        in_specs=[
            pl.BlockSpec((1, W, D), lambda b, nb: (b, 0, 0)),
            pl.BlockSpec((1, BN, D), lambda b, nb: (b, nb, 0)),
        ],
        out_specs=[
            pl.BlockSpec((1, 1, W), lambda b, nb: (b, 0, 0)),
            pl.BlockSpec((1, 1, W), lambda b, nb: (b, 0, 0)),
        ],
        out_shape=[
            jax.ShapeDtypeStruct((B, 1, W), jnp.int32),
            jax.ShapeDtypeStruct((B, 1, W), jnp.int32),
        ],
        scratch_shapes=[
            pltpu.VMEM((W,), jnp.float32),
            pltpu.VMEM((W,), jnp.int32),
        ],
    )


def _make_loss(P, D):
    NW = 32
    PB = P // NW
    NCH = D // 16
    mesh = plsc.VectorSubcoreMesh(core_axis_name="c", subcore_axis_name="s")

    @functools.partial(
        pl.kernel,
        mesh=mesh,
        out_type=jax.ShapeDtypeStruct((P, D), jnp.float32),
        scratch_types=[
            pltpu.VMEM((PB,), jnp.int32),
            pltpu.VMEM((PB, D), jnp.float32),
            pltpu.VMEM((PB, D), jnp.float32),
            pltpu.VMEM((PB, D), jnp.float32),
            pltpu.VMEM((PB, D), jnp.float32),
            pltpu.SemaphoreType.DMA,
            pltpu.SemaphoreType.DMA,
        ],
    )
    def gather_kernel(wp_hbm, bp_hbm, bn_hbm, idx_hbm, out_hbm,
                      idx_v, wp_v, cbp_v, cbn_v, prod_v, sem0, sem1):
        wid = lax.axis_index("s") * 2 + lax.axis_index("c")
        base = wid * PB
        pltpu.sync_copy(idx_hbm.at[pl.ds(base, PB)], idx_v)
        pltpu.sync_copy(wp_hbm.at[pl.ds(base, PB)], wp_v)
        cp0 = pltpu.async_copy(bp_hbm.at[idx_v], cbp_v, sem0)
        cp1 = pltpu.async_copy(bn_hbm.at[idx_v], cbn_v, sem1)
        cp0.wait()
        cp1.wait()

        def pair_body(p, carry):
            for c in range(NCH):
                sl = pl.ds(c * 16, 16)
                prod_v[p, sl] = (wp_v[p, sl] - cbp_v[p, sl]) * cbn_v[p, sl]
            return carry

        lax.fori_loop(0, PB, pair_body, 0)
        pltpu.sync_copy(prod_v, out_hbm.at[pl.ds(base, PB)])

    return gather_kernel


def _make_tc_loss(P, D):
    def body(prod_ref, out_ref):
        dots = jnp.sum(prod_ref[...], axis=1)
        er = jnp.where(dots >= 0.0,
                       jnp.exp(dots) - 1.0,
                       jnp.exp(0.5 * dots) - 1.0)
        out_ref[0, 0] = jnp.sum(er) * (1.0 / P)

    return pl.pallas_call(
        body,
        in_specs=[pl.BlockSpec((P, D), lambda: (0, 0))],
        out_specs=pl.BlockSpec(memory_space=pltpu.SMEM),
        out_shape=jax.ShapeDtypeStruct((1, 1), jnp.float32),
    )


def kernel(waypointslocal, boundarypoints, boundarynormals):
    B, W, D = waypointslocal.shape
    N = boundarypoints.shape[1]
    BN = 2048

    idx3, idxg3 = _make_argmin(B, W, N, D, BN)(waypointslocal, boundarypoints)
    idx = idx3.reshape(B, W)

    prod = _make_loss(B * W, D)(
        waypointslocal.reshape(B * W, D),
        boundarypoints.reshape(B * N, D),
        boundarynormals.reshape(B * N, D),
        idxg3.reshape(B * W),
    )
    loss = _make_tc_loss(B * W, D)(prod)[0, 0]
    return idx, loss


# speedup vs baseline: 1.9485x; 1.2741x over previous
import functools

import jax
import jax.numpy as jnp
from jax import lax
from jax.experimental import pallas as pl
from jax.experimental.pallas import tpu as pltpu
from jax.experimental.pallas import tpu_sc as plsc


def _make_argmin(B, W, N, D, BN):
    NB = N // BN

    def body(wp_ref, bp_ref, idx_ref, idxg_ref, minval, minidx):
        b = pl.program_id(0)
        nb = pl.program_id(1)

        @pl.when(nb == 0)
        def _init():
            minval[...] = jnp.full((W,), jnp.inf, jnp.float32)
            minidx[...] = jnp.zeros((W,), jnp.int32)

        wp = wp_ref[0]
        bp = bp_ref[0]
        w2 = jnp.sum(wp * wp, axis=1)
        b2 = jnp.sum(bp * bp, axis=1)
        crossm2 = lax.dot_general(
            wp * (-2.0), bp, (((1,), (1,)), ((), ())),
            preferred_element_type=jnp.float32)
        s = (w2[:, None] + b2[None, :]) + crossm2
        local_min = jnp.min(s, axis=1)
        idsf = lax.broadcasted_iota(jnp.int32, (W, BN), 1).astype(jnp.float32)
        maskedf = jnp.where(s == local_min[:, None], idsf, jnp.float32(N))
        local_idx = jnp.min(maskedf, axis=1).astype(jnp.int32) + nb * BN
        better = local_min < minval[...]
        minidx[...] = jnp.where(better, local_idx, minidx[...])
        minval[...] = jnp.where(better, local_min, minval[...])

        @pl.when(nb == NB - 1)
        def _fin():
            idx_ref[0, 0] = minidx[...]
            idxg_ref[0, 0] = minidx[...] + b * N

    return pl.pallas_call(
        body,
        grid=(B, NB),
        in_specs=[
            pl.BlockSpec((1, W, D), lambda b, nb: (b, 0, 0)),
            pl.BlockSpec((1, BN, D), lambda b, nb: (b, nb, 0)),
        ],
        out_specs=[
            pl.BlockSpec((1, 1, W), lambda b, nb: (b, 0, 0)),
            pl.BlockSpec((1, 1, W), lambda b, nb: (b, 0, 0)),
        ],
        out_shape=[
            jax.ShapeDtypeStruct((B, 1, W), jnp.int32),
            jax.ShapeDtypeStruct((B, 1, W), jnp.int32),
        ],
        scratch_shapes=[
            pltpu.VMEM((W,), jnp.float32),
            pltpu.VMEM((W,), jnp.int32),
        ],
    )


def _make_loss(P, D):
    NW = 32
    PB = P // NW
    NCH = D // 16
    mesh = plsc.VectorSubcoreMesh(core_axis_name="c", subcore_axis_name="s")

    @functools.partial(
        pl.kernel,
        mesh=mesh,
        out_type=jax.ShapeDtypeStruct((P, D), jnp.float32),
        scratch_types=[
            pltpu.VMEM((PB,), jnp.int32),
            pltpu.VMEM((PB, D), jnp.float32),
            pltpu.VMEM((PB, D), jnp.float32),
            pltpu.VMEM((PB, D), jnp.float32),
            pltpu.VMEM((PB, D), jnp.float32),
            pltpu.SemaphoreType.DMA,
            pltpu.SemaphoreType.DMA,
        ],
    )
    def gather_kernel(wp_hbm, bp_hbm, bn_hbm, idx_hbm, out_hbm,
                      idx_v, wp_v, cbp_v, cbn_v, prod_v, sem0, sem1):
        wid = lax.axis_index("s") * 2 + lax.axis_index("c")
        base = wid * PB
        pltpu.sync_copy(idx_hbm.at[pl.ds(base, PB)], idx_v)
        pltpu.sync_copy(wp_hbm.at[pl.ds(base, PB)], wp_v)
        cp0 = pltpu.async_copy(bp_hbm.at[idx_v], cbp_v, sem0)
        cp1 = pltpu.async_copy(bn_hbm.at[idx_v], cbn_v, sem1)
        cp0.wait()
        cp1.wait()

        def pair_body(p, carry):
            for c in range(NCH):
                sl = pl.ds(c * 16, 16)
                prod_v[p, sl] = (wp_v[p, sl] - cbp_v[p, sl]) * cbn_v[p, sl]
            return carry

        lax.fori_loop(0, PB, pair_body, 0)
        pltpu.sync_copy(prod_v, out_hbm.at[pl.ds(base, PB)])

    return gather_kernel


def _make_tc_loss(P, D):
    def body(prod_ref, out_ref):
        dots = jnp.sum(prod_ref[...], axis=1)
        er = jnp.where(dots >= 0.0,
                       jnp.exp(dots) - 1.0,
                       jnp.exp(0.5 * dots) - 1.0)
        out_ref[0, 0] = jnp.sum(er) * (1.0 / P)

    return pl.pallas_call(
        body,
        in_specs=[pl.BlockSpec((P, D), lambda: (0, 0))],
        out_specs=pl.BlockSpec(memory_space=pltpu.SMEM),
        out_shape=jax.ShapeDtypeStruct((1, 1), jnp.float32),
    )


def kernel(waypointslocal, boundarypoints, boundarynormals):
    B, W, D = waypointslocal.shape
    N = boundarypoints.shape[1]
    BN = 4096

    idx3, idxg3 = _make_argmin(B, W, N, D, BN)(waypointslocal, boundarypoints)
    idx = idx3.reshape(B, W)

    prod = _make_loss(B * W, D)(
        waypointslocal.reshape(B * W, D),
        boundarypoints.reshape(B * N, D),
        boundarynormals.reshape(B * N, D),
        idxg3.reshape(B * W),
    )
    loss = _make_tc_loss(B * W, D)(prod)[0, 0]
    return idx, loss
